# Initial kernel scaffold; baseline (speedup 1.0000x reference)
import jax
import jax.numpy as jnp
from jax.experimental import pallas as pl


def kernel(x, edge_index0, edge_index1, edge_index2, Wq1, Wk1, Wv1, Ak1, Av1, mu1, Wo1, Wq2, Wk2, Wv2, Ak2, Av2, mu2, Wo2):
    raise NotImplementedError("write your pallas kernel here")


# speedup vs baseline: 1.0685x; 1.0685x over previous
import functools
import jax
import jax.numpy as jnp
from jax.experimental import pallas as pl
from jax.experimental.pallas import tpu as pltpu


def _proj_kernel(x_ref, w_ref, out_ref):
    out_ref[...] = jnp.dot(x_ref[...], w_ref[...],
                           preferred_element_type=jnp.float32)


def _project(x, wbig, block=2048):
    n = x.shape[0]
    din = x.shape[1]
    d7 = wbig.shape[1]
    nb = (n + block - 1) // block
    npad = nb * block
    if npad != n:
        x = jnp.pad(x, ((0, npad - n), (0, 0)))
    out = pl.pallas_call(
        _proj_kernel,
        grid=(nb,),
        in_specs=[
            pl.BlockSpec((block, din), lambda i: (i, 0)),
            pl.BlockSpec((din, d7), lambda i: (0, 0)),
        ],
        out_specs=pl.BlockSpec((block, d7), lambda i: (i, 0)),
        out_shape=jax.ShapeDtypeStruct((npad, d7), jnp.float32),
    )(x, wbig)
    return out[:n]


def _hgt_layer(x, edges, Wq, Wk, Wv, Ak, Av, mu, Wo):
    Nn = x.shape[0]
    d = Wq.shape[1]
    scale = 1.0 / (float(d) ** 0.5)
    R = Ak.shape[0]
    wcols = [Wq]
    for r in range(R):
        wcols.append((Wk @ Ak[r]) * (mu[r] * scale))
    for r in range(R):
        wcols.append(Wv @ Av[r])
    wbig = jnp.concatenate(wcols, axis=1)
    tab = _project(x, wbig)
    q = tab[:, :d]
    out = jnp.zeros((Nn, d), dtype=x.dtype)
    for r in range(R):
        src = edges[r][0]
        dst = edges[r][1]
        k_r = tab[:, (1 + r) * d:(2 + r) * d]
        v_r = tab[:, (1 + R + r) * d:(2 + R + r) * d]
        score = jnp.sum(q[dst] * k_r[src], axis=-1)
        m = jax.ops.segment_max(score, dst, num_segments=Nn)
        m = jnp.where(jnp.isfinite(m), m, 0.0)
        e = jnp.exp(score - m[dst])
        ssum = jax.ops.segment_sum(e, dst, num_segments=Nn)
        alpha = e / (ssum[dst] + 1e-9)
        out = out + jax.ops.segment_sum(v_r[src] * alpha[:, None], dst,
                                        num_segments=Nn)
    out = jax.nn.gelu(out) @ Wo
    if out.shape[-1] == x.shape[-1]:
        out = out + x
    return out


def kernel(x, edge_index0, edge_index1, edge_index2,
           Wq1, Wk1, Wv1, Ak1, Av1, mu1, Wo1,
           Wq2, Wk2, Wv2, Ak2, Av2, mu2, Wo2):
    edges = (edge_index0, edge_index1, edge_index2)
    h = _hgt_layer(x, edges, Wq1, Wk1, Wv1, Ak1, Av1, mu1, Wo1)
    h = jax.nn.relu(h)
    y = _hgt_layer(h, edges, Wq2, Wk2, Wv2, Ak2, Av2, mu2, Wo2)
    return y


# speedup vs baseline: 1.0686x; 1.0001x over previous
import jax
import jax.numpy as jnp
from jax.experimental import pallas as pl


def _proj_kernel(x_ref, w_ref, out_ref):
    out_ref[...] = jnp.dot(x_ref[...], w_ref[...],
                           preferred_element_type=jnp.float32)


def _project(x, wbig, block=2048):
    n = x.shape[0]
    din = x.shape[1]
    d7 = wbig.shape[1]
    nb = (n + block - 1) // block
    npad = nb * block
    if npad != n:
        x = jnp.pad(x, ((0, npad - n), (0, 0)))
    out = pl.pallas_call(
        _proj_kernel,
        grid=(nb,),
        in_specs=[
            pl.BlockSpec((block, din), lambda i: (i, 0)),
            pl.BlockSpec((din, d7), lambda i: (0, 0)),
        ],
        out_specs=pl.BlockSpec((block, d7), lambda i: (i, 0)),
        out_shape=jax.ShapeDtypeStruct((npad, d7), jnp.float32),
    )(x, wbig)
    return out[:n]


def _hgt_layer(x, edges, Wq, Wk, Wv, Ak, Av, mu, Wo):
    Nn = x.shape[0]
    d = Wq.shape[1]
    scale = 1.0 / (float(d) ** 0.5)
    R = Ak.shape[0]
    wcols = [Wq]
    for r in range(R):
        wcols.append((Wk @ Ak[r]) * (mu[r] * scale))
    for r in range(R):
        wcols.append(Wv @ Av[r])
    wbig = jnp.concatenate(wcols, axis=1)
    tab = _project(x, wbig)
    q = tab[:, :d]
    out = jnp.zeros((Nn, d), dtype=x.dtype)
    for r in range(R):
        src = edges[r][0]
        dst = edges[r][1]
        k_r = tab[:, (1 + r) * d:(2 + r) * d]
        v_r = tab[:, (1 + R + r) * d:(2 + R + r) * d]
        score = jnp.sum(q[dst] * k_r[src], axis=-1)
        m = jax.ops.segment_max(score, dst, num_segments=Nn)
        m = jnp.where(jnp.isfinite(m), m, 0.0)
        e = jnp.exp(score - m[dst])
        ssum = jax.ops.segment_sum(e, dst, num_segments=Nn)
        alpha = e / (ssum[dst] + 1e-9)
        out = out + jax.ops.segment_sum(v_r[src] * alpha[:, None], dst,
                                        num_segments=Nn)
    out = jax.nn.gelu(out) @ Wo
    if out.shape[-1] == x.shape[-1]:
        out = out + x
    return out


def kernel(x, edge_index0, edge_index1, edge_index2,
           Wq1, Wk1, Wv1, Ak1, Av1, mu1, Wo1,
           Wq2, Wk2, Wv2, Ak2, Av2, mu2, Wo2):
    edges = (edge_index0, edge_index1, edge_index2)
    h = _hgt_layer(x, edges, Wq1, Wk1, Wv1, Ak1, Av1, mu1, Wo1)
    h = jax.nn.relu(h)
    y = _hgt_layer(h, edges, Wq2, Wk2, Wv2, Ak2, Av2, mu2, Wo2)
    return y
